# Initial kernel scaffold; baseline (speedup 1.0000x reference)
import jax
import jax.numpy as jnp
from jax.experimental import pallas as pl


def kernel(x, w1, b1, g1, be1, w2, b2, g2, be2, w3, b3):
    raise NotImplementedError("write your pallas kernel here")


# speedup vs baseline: 9.1411x; 9.1411x over previous
import functools

import jax
import jax.numpy as jnp
from jax import lax
from jax.experimental import pallas as pl
from jax.experimental.pallas import tpu as pltpu
from jax.experimental.pallas import tpu_sc as plsc

K = 13
MW = 15
WN = 2 * MW + 1
NOFF = WN * WN
EPS = 1e-5
B = 4
H = 64
W = 64
C = 8
FD = 64
PW = W + 2
IMROWS = PW * PW
MARGIN = 72
SEG = IMROWS + 2 * MARGIN
TOT = 2 * SEG
PADT = TOT + 2 * MARGIN
NVALID = B * H * W
PG = H + 2 * MW
NPIX = PG * PG
NBAND = H // 8
NW = 32
RPT = B * H * W * K // NW
HALF = RPT // 2


def _mask():
    r = lax.broadcasted_iota(jnp.int32, (TOT, 1), 0)
    q = r % SEG - MARGIN
    row66 = q // PW
    col66 = q % PW
    valid = ((q >= 0) & (q < IMROWS) & (row66 >= 1) & (row66 <= H)
             & (col66 >= 1) & (col66 <= W))
    return valid.astype(jnp.float32)


def _conv_taps(inp_ref, wt_ref, cout, b_ref):
    acc = jnp.zeros((TOT, cout), jnp.float32)
    for t in range(9):
        off = (t // 3 - 1) * PW + (t % 3 - 1)
        sl = inp_ref[MARGIN + off:MARGIN + off + TOT, :]
        acc = acc + lax.dot_general(
            sl, wt_ref[t], (((1,), (0,)), ((), ())),
            preferred_element_type=jnp.float32)
    return acc + b_ref[...][None, :]


def _conv_body(cout, xp_ref, wt_ref, b_ref, out_ref):
    out_ref[...] = _conv_taps(xp_ref, wt_ref, cout, b_ref)


def _knn_body(eq_ref, ep_ref, out_ref, dsc_ref):
    y0 = pl.program_id(1) * 8
    eq = eq_ref[0]

    def dy_body(dy, carry):
        epsl = ep_ref[0, :, pl.ds(y0 + dy, 8), :]
        for dx in range(WN):
            sl = lax.slice(epsl, (0, 0, dx), (C, 8, dx + W))
            diff = eq - sl
            dsc_ref[dy * WN + dx] = jnp.sum(diff * diff, axis=0)
        return carry

    lax.fori_loop(0, WN, dy_body, 0)
    dsc_ref[pl.ds(NOFF, 1024 - NOFF)] = jnp.full(
        (1024 - NOFF, 8, 64), 1e30, jnp.float32)

    dmat = dsc_ref[...]
    iota_o = lax.broadcasted_iota(jnp.int32, (1024, 8, 64), 0)
    ytile = lax.broadcasted_iota(jnp.int32, (8, 64), 0) + y0
    xtile = lax.broadcasted_iota(jnp.int32, (8, 64), 1)
    for k in range(K + 1):
        mval = jnp.min(dmat, axis=0)
        cand = jnp.where(dmat == mval[None], iota_o, 1024)
        idxk = jnp.min(cand, axis=0)
        pflat = (idxk // WN + ytile) * PG + (idxk % WN + xtile)
        out_ref[0, 0, k] = pflat
        dmat = jnp.where(iota_o == idxk[None], jnp.float32(1e30), dmat)


def _sc_gather_body(tab_hbm, idx_hbm, out_hbm, tabv, idxv, outv):
    wid = lax.axis_index("s") * 2 + lax.axis_index("c")
    b = wid // 8
    pltpu.sync_copy(tab_hbm.at[b], tabv)
    pltpu.sync_copy(idx_hbm.at[wid], idxv)

    def do_half(h):
        def body(i, carry):
            idxs = idxv[pl.ds(h * HALF + i * 16, 16)]
            base = idxs * C
            for c in range(C):
                g = plsc.load_gather(tabv, [base + c])
                outv[c, pl.ds(i * 16, 16)] = g
            return carry

        lax.fori_loop(0, HALF // 16, body, 0)
        pltpu.sync_copy(outv, out_hbm.at[wid, h])

    do_half(0)
    do_half(1)


@functools.lru_cache(maxsize=1)
def _sc_gather_call():
    return functools.partial(
        pl.kernel,
        mesh=plsc.VectorSubcoreMesh(core_axis_name="c", subcore_axis_name="s"),
        compiler_params=pltpu.CompilerParams(needs_layout_passes=False),
        out_type=jax.ShapeDtypeStruct((NW, 2, C, HALF), jnp.float32),
        scratch_types=[
            pltpu.VMEM((NPIX * C,), jnp.float32),
            pltpu.VMEM((RPT,), jnp.int32),
            pltpu.VMEM((C, HALF), jnp.float32),
        ],
    )(_sc_gather_body)


def _conv(xp, wt, b, cout):
    return pl.pallas_call(
        functools.partial(_conv_body, cout),
        out_shape=jax.ShapeDtypeStruct((TOT, cout), jnp.float32),
    )(xp, wt, b)


def _pack_flat(hwc_pad, nc):
    hs = (hwc_pad.reshape(2, 2, IMROWS, nc).transpose(1, 2, 0, 3)
          .reshape(2, IMROWS, 2 * nc))
    seg = jnp.pad(hs, ((0, 0), (MARGIN, MARGIN), (0, 0)))
    return jnp.pad(seg.reshape(TOT, 2 * nc), ((MARGIN, MARGIN), (0, 0)))


def _unpack_nchw(out, nc):
    e = (out.reshape(2, SEG, 2 * nc)[:, MARGIN:MARGIN + IMROWS]
         .reshape(2, PW, PW, 2, nc).transpose(3, 0, 1, 2, 4)
         .reshape(B, PW, PW, nc)[:, 1:1 + H, 1:1 + W, :])
    return jnp.transpose(e, (0, 3, 1, 2))


def _knn(eqt, ept):
    return pl.pallas_call(
        _knn_body,
        grid=(B, NBAND),
        in_specs=[
            pl.BlockSpec((1, C, 8, W), lambda b, i: (b, 0, i, 0)),
            pl.BlockSpec((1, C, PG, PG), lambda b, i: (b, 0, 0, 0)),
        ],
        out_specs=pl.BlockSpec((1, 1, K + 1, 8, W),
                               lambda b, i: (b, i, 0, 0, 0)),
        out_shape=jax.ShapeDtypeStruct((B, NBAND, K + 1, 8, W), jnp.int32),
        scratch_shapes=[pltpu.VMEM((1024, 8, W), jnp.float32)],
    )(eqt, ept)


def kernel(x, w1, b1, g1, be1, w2, b2, g2, be2, w3, b3):
    xhwc = jnp.transpose(x, (0, 2, 3, 1))
    xp = _pack_flat(jnp.pad(xhwc, ((0, 0), (1, 1), (1, 1), (0, 0))), 3)

    def blockdiag(wt, ci, co):
        z = jnp.zeros((9, 2 * ci, 2 * co), jnp.float32)
        z = z.at[:, :ci, :co].set(wt)
        return z.at[:, ci:, co:].set(wt)

    wt1 = blockdiag(jnp.transpose(w1, (2, 3, 1, 0)).reshape(9, 3, FD), 3, FD)
    wt2 = blockdiag(jnp.transpose(w2, (2, 3, 1, 0)).reshape(9, FD, FD), FD, FD)
    wt3 = blockdiag(jnp.transpose(w3, (2, 3, 1, 0)).reshape(9, FD, C), FD, C)
    dup = lambda v: jnp.concatenate([v, v])

    def bn_relu_repack(h, g, be):
        hn = _unpack_nchw(h, FD)
        m = jnp.mean(hn, axis=(0, 2, 3), keepdims=True)
        v = jnp.var(hn, axis=(0, 2, 3), keepdims=True)
        r = jax.nn.relu(g[None, :, None, None] * (hn - m)
                        / jnp.sqrt(v + EPS) + be[None, :, None, None])
        rhwc = jnp.transpose(r, (0, 2, 3, 1))
        return _pack_flat(jnp.pad(rhwc, ((0, 0), (1, 1), (1, 1), (0, 0))), FD)

    h1 = _conv(xp, wt1, dup(b1), 2 * FD)
    h2 = _conv(bn_relu_repack(h1, g1, be1), wt2, dup(b2), 2 * FD)
    e_full = _conv(bn_relu_repack(h2, g2, be2), wt3, dup(b3), 2 * C)
    e_hwc = (e_full.reshape(2, SEG, 2 * C)[:, MARGIN:MARGIN + IMROWS]
             .reshape(2, PW, PW, 2, C).transpose(3, 0, 1, 2, 4)
             .reshape(B, PW, PW, C)[:, 1:1 + H, 1:1 + W, :])
    e_nchw = jnp.transpose(e_hwc, (0, 3, 1, 2))

    ept = jnp.pad(e_nchw, ((0, 0), (0, 0), (MW, MW), (MW, MW)),
                  constant_values=1e4)
    pidx = _knn(e_nchw, ept)

    idx_hwk = (jnp.transpose(pidx, (0, 2, 1, 3, 4))
               .reshape(B, K + 1, H, W)[:, 1:]
               .transpose(0, 2, 3, 1))
    idx_sc = idx_hwk.reshape(NW, RPT)
    tab = jnp.pad(e_hwc, ((0, 0), (MW, MW), (MW, MW), (0, 0)),
                  constant_values=1e4).reshape(B, NPIX * C)
    g = _sc_gather_call()(tab, idx_sc)

    z = (g.reshape(B, 8, 2, C, HALF // K, K)
         .transpose(0, 5, 3, 1, 2, 4)
         .reshape(B, K * C, H, W))
    return jnp.concatenate([e_nchw, z], axis=1)
